# baseline (speedup 1.0000x reference)
import jax
import jax.numpy as jnp
from jax.experimental import pallas as pl


def _psm_kernel(x_ref, s_ref, out_ref, ns_ref):
    ns = x_ref[...] + s_ref[...]
    ns_ref[...] = ns
    e = jnp.exp(ns)
    denom = jnp.sum(e, axis=-1, keepdims=True)
    out_ref[...] = e * (2.0 / denom)


def kernel(x, state):
    M, N = x.shape
    R = 8
    bs = pl.BlockSpec((R, N), lambda i: (i, 0))
    out, ns = pl.pallas_call(
        _psm_kernel,
        grid=(M // R,),
        in_specs=[bs, bs],
        out_specs=[bs, bs],
        out_shape=[
            jax.ShapeDtypeStruct((M, N), x.dtype),
            jax.ShapeDtypeStruct((M, N), x.dtype),
        ],
    )(x, state)
    return (out, ns)
